# baseline (speedup 1.0000x reference)
import functools

import jax
import jax.numpy as jnp
from jax import lax
from jax.experimental import pallas as pl
from jax.experimental.pallas import tpu as pltpu
from jax.experimental.pallas import tpu_sc as plsc

_NC, _NS, _L = 2, 16, 16
_NW = _NC * _NS
_CH = 32


def _make_sc_add(B, S, D):
    rows_per_w = S // _NW
    n_chunks = rows_per_w // _CH
    mesh = plsc.VectorSubcoreMesh(core_axis_name="c", subcore_axis_name="s")

    @functools.partial(
        pl.kernel,
        out_type=jax.ShapeDtypeStruct((B, S, D), jnp.float32),
        mesh=mesh,
        scratch_types=[
            pltpu.VMEM((_CH, D), jnp.float32),
            pltpu.VMEM((_CH, D), jnp.float32),
        ],
    )
    def sc_add(x_hbm, emb_hbm, out_hbm, ebuf, xbuf):
        wid = lax.axis_index("s") * _NC + lax.axis_index("c")
        base = wid * rows_per_w

        def chunk_body(c, carry):
            r0 = base + c * _CH
            pltpu.sync_copy(emb_hbm.at[pl.ds(r0, _CH)], ebuf)
            for b in range(B):
                pltpu.sync_copy(x_hbm.at[b, pl.ds(r0, _CH)], xbuf)

                def row_body(r, carry2):
                    for j in range(D // _L):
                        e = ebuf[r, pl.ds(j * _L, _L)]
                        plsc.addupdate(xbuf.at[r, pl.ds(j * _L, _L)], e)
                    return carry2

                lax.fori_loop(0, _CH, row_body, 0)
                pltpu.sync_copy(xbuf, out_hbm.at[b, pl.ds(r0, _CH)])
            return carry

        lax.fori_loop(0, n_chunks, chunk_body, 0)

    return sc_add


def kernel(x, emb_table):
    B, S, D = x.shape
    return _make_sc_add(B, S, D)(x, emb_table[:S])


# speedup vs baseline: 1.9171x; 1.9171x over previous
import functools

import jax
import jax.numpy as jnp
from jax import lax
from jax.experimental import pallas as pl
from jax.experimental.pallas import tpu as pltpu
from jax.experimental.pallas import tpu_sc as plsc

_NC, _NS, _L = 2, 16, 16
_NW = _NC * _NS
_CH = 16


def _make_sc_add(B, S, D):
    rows_per_w = S // _NW
    n_chunks = rows_per_w // _CH
    mesh = plsc.VectorSubcoreMesh(core_axis_name="c", subcore_axis_name="s")

    @functools.partial(
        pl.kernel,
        out_type=jax.ShapeDtypeStruct((B, S, D), jnp.float32),
        mesh=mesh,
        scratch_types=(
            [pltpu.VMEM((_CH, D), jnp.float32)]
            + [pltpu.VMEM((_CH, D), jnp.float32) for _ in range(B)]
            + [pltpu.SemaphoreType.DMA for _ in range(1 + 2 * B)]
        ),
    )
    def sc_add(x_hbm, emb_hbm, out_hbm, *bufs):
        ebuf = bufs[0]
        xbufs = bufs[1 : 1 + B]
        esem = bufs[1 + B]
        xisems = bufs[2 + B : 2 + 2 * B]
        xosems = bufs[2 + 2 * B : 2 + 3 * B]

        wid = lax.axis_index("s") * _NC + lax.axis_index("c")
        base = wid * rows_per_w
        last = n_chunks - 1

        def start_e(c):
            return pltpu.async_copy(emb_hbm.at[pl.ds(base + c * _CH, _CH)], ebuf, esem)

        def start_xin(c, b):
            return pltpu.async_copy(
                x_hbm.at[b, pl.ds(base + c * _CH, _CH)], xbufs[b], xisems[b]
            )

        def start_xout(c, b):
            return pltpu.async_copy(
                xbufs[b], out_hbm.at[b, pl.ds(base + c * _CH, _CH)], xosems[b]
            )

        def wait_e():
            pltpu.make_async_copy(emb_hbm.at[pl.ds(base, _CH)], ebuf, esem).wait()

        def wait_xin(b):
            pltpu.make_async_copy(
                x_hbm.at[b, pl.ds(base, _CH)], xbufs[b], xisems[b]
            ).wait()

        def wait_xout(b):
            pltpu.make_async_copy(
                xbufs[b], out_hbm.at[b, pl.ds(base, _CH)], xosems[b]
            ).wait()

        def accumulate(xb):
            @plsc.parallel_loop(0, _CH)
            def _(r):
                for j in range(D // _L):
                    e = ebuf[r, pl.ds(j * _L, _L)]
                    plsc.addupdate(xb.at[r, pl.ds(j * _L, _L)], e)

        start_e(0)
        for b in range(B):
            start_xin(0, b)

        def chunk_body(c, carry):
            cn = jnp.minimum(c + 1, last)
            wait_e()
            for b in range(B):
                wait_xin(b)
                accumulate(xbufs[b])
                start_xout(c, b)
                if b > 0:
                    wait_xout(b - 1)
                    start_xin(cn, b - 1)
            start_e(cn)
            wait_xout(B - 1)
            start_xin(cn, B - 1)
            return carry

        lax.fori_loop(0, n_chunks, chunk_body, 0)

        wait_e()
        for b in range(B):
            wait_xin(b)

    return sc_add


def kernel(x, emb_table):
    B, S, D = x.shape
    return _make_sc_add(B, S, D)(x, emb_table[:S])


# speedup vs baseline: 1.9408x; 1.0124x over previous
import functools

import jax
import jax.numpy as jnp
from jax import lax
from jax.experimental import pallas as pl
from jax.experimental.pallas import tpu as pltpu
from jax.experimental.pallas import tpu_sc as plsc

_NC, _NS, _L = 2, 16, 16
_NW = _NC * _NS
_CH = 16


def _make_sc_add(B, S, D):
    rows_per_w = S // _NW
    n_chunks = rows_per_w // _CH
    mesh = plsc.VectorSubcoreMesh(core_axis_name="c", subcore_axis_name="s")

    @functools.partial(
        pl.kernel,
        out_type=jax.ShapeDtypeStruct((B, S, D), jnp.float32),
        mesh=mesh,
        scratch_types=(
            [pltpu.VMEM((2, _CH, D), jnp.float32)]
            + [pltpu.VMEM((_CH, D), jnp.float32) for _ in range(B)]
            + [pltpu.SemaphoreType.DMA for _ in range(1 + 2 * B)]
        ),
    )
    def sc_add(x_hbm, emb_hbm, out_hbm, *bufs):
        ebuf = bufs[0]
        xbufs = bufs[1 : 1 + B]
        esem = bufs[1 + B]
        xisems = bufs[2 + B : 2 + 2 * B]
        xosems = bufs[2 + 2 * B : 2 + 3 * B]

        wid = lax.axis_index("s") * _NC + lax.axis_index("c")
        base = wid * rows_per_w
        last = n_chunks - 1

        def start_e(c_addr, par):
            return pltpu.async_copy(
                emb_hbm.at[pl.ds(base + c_addr * _CH, _CH)], ebuf.at[par], esem
            )

        def start_xin(c, b):
            return pltpu.async_copy(
                x_hbm.at[b, pl.ds(base + c * _CH, _CH)], xbufs[b], xisems[b]
            )

        def start_xout(c, b):
            return pltpu.async_copy(
                xbufs[b], out_hbm.at[b, pl.ds(base + c * _CH, _CH)], xosems[b]
            )

        def wait_e():
            pltpu.make_async_copy(
                emb_hbm.at[pl.ds(base, _CH)], ebuf.at[0], esem
            ).wait()

        def wait_xin(b):
            pltpu.make_async_copy(
                x_hbm.at[b, pl.ds(base, _CH)], xbufs[b], xisems[b]
            ).wait()

        def wait_xout(b):
            pltpu.make_async_copy(
                xbufs[b], out_hbm.at[b, pl.ds(base, _CH)], xosems[b]
            ).wait()

        def accumulate(xb, par):
            @plsc.parallel_loop(0, _CH, unroll=2)
            def _(r):
                for j in range(D // _L):
                    e = ebuf[par, r, pl.ds(j * _L, _L)]
                    plsc.addupdate(xb.at[r, pl.ds(j * _L, _L)], e)

        start_e(0, 0)
        for b in range(B):
            start_xin(0, b)

        def chunk_body(c, carry):
            cn = jnp.minimum(c + 1, last)
            par = c % 2
            wait_e()
            start_e(cn, (c + 1) % 2)
            for b in range(B):
                wait_xin(b)
                accumulate(xbufs[b], par)
                start_xout(c, b)
                if b > 0:
                    wait_xout(b - 1)
                    start_xin(cn, b - 1)
            wait_xout(B - 1)
            start_xin(cn, B - 1)
            return carry

        lax.fori_loop(0, n_chunks, chunk_body, 0)

        wait_e()
        for b in range(B):
            wait_xin(b)

    return sc_add


def kernel(x, emb_table):
    B, S, D = x.shape
    return _make_sc_add(B, S, D)(x, emb_table[:S])


# speedup vs baseline: 2.2341x; 1.1511x over previous
import functools

import jax
import jax.numpy as jnp
from jax import lax
from jax.experimental import pallas as pl
from jax.experimental.pallas import tpu as pltpu
from jax.experimental.pallas import tpu_sc as plsc

_NC, _NS, _L = 2, 16, 16
_NW = _NC * _NS
_CH = 8


def _make_sc_add(B, S, D):
    rows_per_w = S // _NW
    n_chunks = rows_per_w // _CH
    chunk_bytes = _CH * D * 4
    mesh = plsc.VectorSubcoreMesh(core_axis_name="c", subcore_axis_name="s")

    @functools.partial(
        pl.kernel,
        out_type=jax.ShapeDtypeStruct((B, S, D), jnp.float32),
        mesh=mesh,
        scratch_types=[
            pltpu.VMEM((2, _CH, D), jnp.float32),
            pltpu.VMEM((2, B, _CH, D), jnp.float32),
            pltpu.SemaphoreType.DMA,
            pltpu.SemaphoreType.DMA,
            pltpu.SemaphoreType.DMA,
        ],
    )
    def sc_add(x_hbm, emb_hbm, out_hbm, ebuf, xbuf, esem, xisem, xosem):
        wid = lax.axis_index("s") * _NC + lax.axis_index("c")
        base = wid * rows_per_w
        last = n_chunks - 1

        def start_e(c_addr, par):
            pltpu.async_copy(
                emb_hbm.at[pl.ds(base + c_addr * _CH, _CH)], ebuf.at[par], esem
            )

        def start_xin(c_addr, b, gen):
            pltpu.async_copy(
                x_hbm.at[b, pl.ds(base + c_addr * _CH, _CH)], xbuf.at[gen, b], xisem
            )

        def start_xout(c_addr, b, gen):
            pltpu.async_copy(
                xbuf.at[gen, b], out_hbm.at[b, pl.ds(base + c_addr * _CH, _CH)], xosem
            )

        def wait_e():
            pltpu.make_async_copy(
                emb_hbm.at[pl.ds(base, _CH)], ebuf.at[0], esem
            ).wait()

        def wait_xin():
            pltpu.make_async_copy(
                x_hbm.at[0, pl.ds(base, _CH)], xbuf.at[0, 0], xisem
            ).wait()

        def wait_xout():
            pltpu.make_async_copy(
                xbuf.at[0, 0], out_hbm.at[0, pl.ds(base, _CH)], xosem
            ).wait()

        def accumulate(gen, par):
            @plsc.parallel_loop(0, _CH, unroll=2)
            def _(r):
                for j in range(D // _L):
                    e = ebuf[par, r, pl.ds(j * _L, _L)]
                    for b in range(B):
                        plsc.addupdate(xbuf.at[gen, b, r, pl.ds(j * _L, _L)], e)

        start_e(0, 0)
        for b in range(B):
            start_xin(0, b, 0)

        def chunk_body(c, carry):
            cn = jnp.minimum(c + 1, last)
            gen = c % 2
            gen_n = (c + 1) % 2
            wait_e()
            start_e(cn, gen_n)
            for _ in range(B):
                wait_xin()
            @pl.when(c > 0)
            def _():
                for _ in range(B):
                    wait_xout()

            for b in range(B):
                start_xin(cn, b, gen_n)
            accumulate(gen, gen)
            for b in range(B):
                start_xout(c, b, gen)
            return carry

        lax.fori_loop(0, n_chunks, chunk_body, 0)

        wait_e()
        for _ in range(B):
            wait_xin()
            wait_xout()

    return sc_add


def kernel(x, emb_table):
    B, S, D = x.shape
    return _make_sc_add(B, S, D)(x, emb_table[:S])
